# Initial kernel scaffold; baseline (speedup 1.0000x reference)
import jax
import jax.numpy as jnp
from jax.experimental import pallas as pl


def kernel(hidden_states, W_up, W_down, W_q, keys, up_embed, down_embed):
    raise NotImplementedError("write your pallas kernel here")


# speedup vs baseline: 8.0037x; 8.0037x over previous
import functools

import jax
import jax.numpy as jnp
from jax import lax
from jax.experimental import pallas as pl
from jax.experimental.pallas import tpu as pltpu
from jax.experimental.pallas import tpu_sc as plsc

HIDDEN = 1024
SHARED = 4096
PRIVATE = 1024
N_EXPERTS = 4096
N_HEADS = 4
K_PER_HEAD = 4
NUM_KEYS = 64
DHALF = PRIVATE // 2
T = 2048

TT = 256
NEG = float("-inf")


def _h_body(x_ref, wup_ref, wdn_ref, h_ref):
    xb = x_ref[...].astype(jnp.bfloat16)
    mid = jnp.dot(xb, wup_ref[...], preferred_element_type=jnp.float32)
    midb = jax.nn.silu(mid).astype(jnp.bfloat16)
    h_ref[...] = jnp.dot(midb, wdn_ref[...], preferred_element_type=jnp.float32)


def _stage_h(x, wup_b, wdn_b):
    return pl.pallas_call(
        _h_body,
        grid=(T // TT,),
        in_specs=[
            pl.BlockSpec((TT, HIDDEN), lambda i: (i, 0)),
            pl.BlockSpec((HIDDEN, SHARED), lambda i: (0, 0)),
            pl.BlockSpec((SHARED, PRIVATE), lambda i: (0, 0)),
        ],
        out_specs=pl.BlockSpec((TT, PRIVATE), lambda i: (i, 0)),
        out_shape=jax.ShapeDtypeStruct((T, PRIVATE), jnp.float32),
    )(x, wup_b, wdn_b)


def _top4(s):
    n = s.shape[-1]
    iota = lax.broadcasted_iota(jnp.int32, s.shape, len(s.shape) - 1)
    vals, poss = [], []
    for _ in range(K_PER_HEAD):
        m = jnp.max(s, axis=-1, keepdims=True)
        hit = s == m
        pos = jnp.min(jnp.where(hit, iota, n), axis=-1, keepdims=True)
        vals.append(m)
        poss.append(pos)
        s = jnp.where(iota == pos, NEG, s)
    return vals, poss


def _route_body(h_ref, wq_ref, keys_ref, up_ref, h2_ref, idx_ref, pw_ref):
    hb = h_ref[...].astype(jnp.bfloat16)
    q = jnp.dot(hb, wq_ref[...], preferred_element_type=jnp.float32)
    h2_ref[...] = lax.dot_general(
        hb, up_ref[...], (((1,), (1,)), ((), ())),
        preferred_element_type=jnp.float32)

    idx_cols, pw_cols = [], []
    for hh in range(N_HEADS):
        per_p = []
        for p in range(2):
            start = (p * N_HEADS + hh) * DHALF
            qp = q[:, start:start + DHALF].astype(jnp.bfloat16)
            sim = jnp.dot(qp, keys_ref[p, hh], preferred_element_type=jnp.float32)
            per_p.append(_top4(sim))
        (vx, ix), (vy, iy) = per_p
        all_s = jnp.concatenate(
            [vx[i] + vy[j] for i in range(4) for j in range(4)], axis=1)
        all_i = jnp.concatenate(
            [ix[i] * NUM_KEYS + iy[j] for i in range(4) for j in range(4)], axis=1)
        iota16 = lax.broadcasted_iota(jnp.int32, all_s.shape, 1)
        s = all_s
        svals, eidx = [], []
        for _ in range(K_PER_HEAD):
            m = jnp.max(s, axis=-1, keepdims=True)
            hit = s == m
            pos = jnp.min(jnp.where(hit, iota16, 16), axis=-1, keepdims=True)
            e = jnp.sum(jnp.where(iota16 == pos, all_i, 0), axis=-1, keepdims=True)
            svals.append(m)
            eidx.append(e)
            s = jnp.where(iota16 == pos, NEG, s)
        sc = jnp.concatenate(svals, axis=1)
        mx = jnp.max(sc, axis=-1, keepdims=True)
        ex = jnp.exp(sc - mx)
        pw = ex / jnp.sum(ex, axis=-1, keepdims=True)
        idx_cols.extend(eidx)
        pw_cols.append(pw)
    idx_ref[...] = jnp.concatenate(idx_cols, axis=1)
    pw_ref[...] = jnp.concatenate(pw_cols, axis=1)


def _stage_route(h, wq_b, keys_b, up_b):
    return pl.pallas_call(
        _route_body,
        grid=(T // TT,),
        in_specs=[
            pl.BlockSpec((TT, PRIVATE), lambda i: (i, 0)),
            pl.BlockSpec((PRIVATE, 2 * N_HEADS * DHALF), lambda i: (0, 0)),
            pl.BlockSpec((2, N_HEADS, DHALF, NUM_KEYS), lambda i: (0, 0, 0, 0)),
            pl.BlockSpec((N_EXPERTS, PRIVATE), lambda i: (0, 0)),
        ],
        out_specs=[
            pl.BlockSpec((TT, N_EXPERTS), lambda i: (i, 0)),
            pl.BlockSpec((TT, 16), lambda i: (i, 0)),
            pl.BlockSpec((TT, 16), lambda i: (i, 0)),
        ],
        out_shape=[
            jax.ShapeDtypeStruct((T, N_EXPERTS), jnp.float32),
            jax.ShapeDtypeStruct((T, 16), jnp.int32),
            jax.ShapeDtypeStruct((T, 16), jnp.float32),
        ],
    )(h, wq_b, keys_b, up_b)


_NC, _NS = 2, 16
_NW = _NC * _NS
_TPW = T // _NW
_TBLK = 8


def _sc_combine(h2, idx, pw):
    mesh = plsc.VectorSubcoreMesh(core_axis_name="c", subcore_axis_name="s")

    @functools.partial(
        pl.kernel,
        mesh=mesh,
        out_type=jax.ShapeDtypeStruct((T, N_EXPERTS), jnp.float32),
        compiler_params=pltpu.CompilerParams(needs_layout_passes=False),
        scratch_types=[
            pltpu.VMEM((_TPW, 16), jnp.int32),
            pltpu.VMEM((_TPW, 16), jnp.float32),
            pltpu.VMEM((_TBLK, N_EXPERTS), jnp.float32),
            pltpu.VMEM((_TBLK, N_EXPERTS), jnp.float32),
        ],
    )
    def sck(h2_hbm, idx_hbm, pw_hbm, s_hbm, idx_v, pw_v, hbuf, sbuf):
        wid = lax.axis_index("s") * _NC + lax.axis_index("c")
        base = wid * _TPW
        pltpu.sync_copy(idx_hbm.at[pl.ds(base, _TPW)], idx_v)
        pltpu.sync_copy(pw_hbm.at[pl.ds(base, _TPW)], pw_v)

        def zero_body(i, carry):
            r = i // (N_EXPERTS // 16)
            c = (i % (N_EXPERTS // 16)) * 16
            sbuf[r, pl.ds(c, 16)] = jnp.zeros((16,), jnp.float32)
            return carry

        lax.fori_loop(0, _TBLK * (N_EXPERTS // 16), zero_body, 0)

        def blk_body(bb, carry):
            t0 = base + bb * _TBLK
            pltpu.sync_copy(h2_hbm.at[pl.ds(t0, _TBLK)], hbuf)
            lane = lax.iota(jnp.int32, 16)
            for i in range(_TBLK):
                tl = bb * _TBLK + i
                idxrow = idx_v[tl, :]
                pwrow = pw_v[tl, :]
                rowi = jnp.full((16,), i, jnp.int32)
                x16 = plsc.load_gather(hbuf, [rowi, idxrow])
                w = x16 * pwrow / (1.0 + jnp.exp(-x16))
                for hh in range(N_HEADS):
                    plsc.addupdate_scatter(
                        sbuf, [rowi, idxrow], w, mask=(lane // 4) == hh)
            pltpu.sync_copy(sbuf, s_hbm.at[pl.ds(t0, _TBLK)])
            for i in range(_TBLK):
                tl = bb * _TBLK + i
                idxrow = idx_v[tl, :]
                rowi = jnp.full((16,), i, jnp.int32)
                plsc.store_scatter(sbuf, [rowi, idxrow],
                                   jnp.zeros((16,), jnp.float32))
            return carry

        lax.fori_loop(0, _TPW // _TBLK, blk_body, 0)

    return sck(h2, idx, pw)


def _out_body(s_ref, down_ref, o_ref):
    sb = s_ref[...].astype(jnp.bfloat16)
    o_ref[...] = jnp.dot(sb, down_ref[...], preferred_element_type=jnp.float32)


def _stage_out(s, down_b):
    return pl.pallas_call(
        _out_body,
        grid=(T // TT,),
        in_specs=[
            pl.BlockSpec((TT, N_EXPERTS), lambda i: (i, 0)),
            pl.BlockSpec((N_EXPERTS, HIDDEN), lambda i: (0, 0)),
        ],
        out_specs=pl.BlockSpec((TT, HIDDEN), lambda i: (i, 0)),
        out_shape=jax.ShapeDtypeStruct((T, HIDDEN), jnp.float32),
    )(s, down_b)


def kernel(hidden_states, W_up, W_down, W_q, keys, up_embed, down_embed):
    x = hidden_states.reshape(T, HIDDEN)
    wup_b = W_up.astype(jnp.bfloat16)
    wdn_b = W_down.astype(jnp.bfloat16)
    wq_b = W_q.astype(jnp.bfloat16)
    keys_b = keys.transpose(2, 0, 3, 1).astype(jnp.bfloat16)
    up_b = up_embed.astype(jnp.bfloat16)
    down_b = down_embed.astype(jnp.bfloat16)

    h = _stage_h(x, wup_b, wdn_b)
    h2, idx, pw = _stage_route(h, wq_b, keys_b, up_b)
    s = _sc_combine(h2, idx, pw)
    out = _stage_out(s, down_b)
    return out.reshape(1, T, HIDDEN)
